# Initial kernel scaffold; baseline (speedup 1.0000x reference)
import jax
import jax.numpy as jnp
from jax.experimental import pallas as pl


def kernel(x, edge_index, edge_weight, train_fts_id, W, M):
    raise NotImplementedError("write your pallas kernel here")


# speedup vs baseline: 3.3609x; 3.3609x over previous
import functools

import jax
import jax.numpy as jnp
from jax import lax
from jax.experimental import pallas as pl
from jax.experimental.pallas import tpu as pltpu
from jax.experimental.pallas import tpu_sc as plsc

N = 10000
E = 320000
D = 128

NC = 2
NS = 16
CHUNK = 128
NCH = 157
PER_TILE = NCH * CHUNK
E_PAD = PER_TILE * NS
ROWS_PER_TILE = 640
N_PAD = ROWS_PER_TILE * NS

PRO_BLK = 400
EPI_BLK = 512
NT_PAD = 1024


def _pro_body(ids_ref, m_ref, x_ref, meff_ref, mx_ref):
    base = pl.program_id(0) * PRO_BLK
    rows = base + lax.broadcasted_iota(jnp.int32, (PRO_BLK, 1), 0)
    ids = ids_ref[...]
    hit = jnp.zeros((PRO_BLK, 1), dtype=jnp.bool_)
    for j in range(NT_PAD // 128):
        hit = hit | jnp.any(rows == ids[j:j + 1, :], axis=1, keepdims=True)
    meff = jnp.where(hit, 1.0, jax.nn.sigmoid(m_ref[...]))
    meff_ref[...] = meff
    mx_ref[...] = meff * x_ref[...]


def _prologue(train_ids_pad, m, x):
    return pl.pallas_call(
        _pro_body,
        grid=(N // PRO_BLK,),
        in_specs=[
            pl.BlockSpec((NT_PAD // 128, 128), lambda i: (0, 0)),
            pl.BlockSpec((PRO_BLK, D), lambda i: (i, 0)),
            pl.BlockSpec((PRO_BLK, D), lambda i: (i, 0)),
        ],
        out_specs=[
            pl.BlockSpec((PRO_BLK, D), lambda i: (i, 0)),
            pl.BlockSpec((PRO_BLK, D), lambda i: (i, 0)),
        ],
        out_shape=[
            jax.ShapeDtypeStruct((N, D), jnp.float32),
            jax.ShapeDtypeStruct((N, D), jnp.float32),
        ],
    )(train_ids_pad, m, x)


def _sc_body(meff_hbm, mx_hbm, sidx_hbm, didx_hbm, w_hbm, dsum_hbm, hsum_hbm,
             sidx128, didx128, w128, rows_v, acc_sh, sem):
    cid = lax.axis_index("c")
    tid = lax.axis_index("s")
    tbase = tid * PER_TILE

    def _zero_row(i, _):
        for j in range(D // 16):
            rows_v[i, pl.ds(j * 16, 16)] = jnp.zeros((16,), jnp.float32)
        return 0
    lax.fori_loop(0, CHUNK, _zero_row, 0)
    for k in range(ROWS_PER_TILE // CHUNK):
        pltpu.sync_copy(
            rows_v, acc_sh.at[pl.ds(tid * ROWS_PER_TILE + k * CHUNK, CHUNK)])
    plsc.subcore_barrier()

    def _chunk(c, carry, scaled):
        del carry
        off = tbase + c * CHUNK
        pltpu.sync_copy(sidx_hbm.at[pl.ds(off, CHUNK)], sidx128)
        pltpu.sync_copy(didx_hbm.at[pl.ds(off, CHUNK)], didx128)
        if scaled:
            pltpu.sync_copy(w_hbm.at[pl.ds(off, CHUNK)], w128)
        table = mx_hbm if scaled else meff_hbm
        pltpu.async_copy(table.at[sidx128], rows_v, sem).wait()
        if scaled:
            def _scale_grp(g, _):
                wv = w128[pl.ds(g * 16, 16)]
                for l in range(16):
                    wl = wv[l]
                    e = g * 16 + l
                    for j in range(D // 16):
                        sl = pl.ds(j * 16, 16)
                        rows_v[e, sl] = rows_v[e, sl] * wl
                return 0
            lax.fori_loop(0, CHUNK // 16, _scale_grp, 0)
        pltpu.sync_copy(rows_v, acc_sh.at[didx128], add=True)
        return c + 1

    @pl.when(cid == 0)
    def _():
        lax.fori_loop(0, NCH, lambda c, a: _chunk(c, a, scaled=False), 0)

    @pl.when(cid == 1)
    def _():
        lax.fori_loop(0, NCH, lambda c, a: _chunk(c, a, scaled=True), 0)

    plsc.subcore_barrier()

    out_slice = pl.ds(tid * ROWS_PER_TILE, ROWS_PER_TILE)

    @pl.when(cid == 0)
    def _():
        pltpu.sync_copy(acc_sh.at[out_slice], dsum_hbm.at[out_slice])

    @pl.when(cid == 1)
    def _():
        pltpu.sync_copy(acc_sh.at[out_slice], hsum_hbm.at[out_slice])


def _segment_sums(meff, mx, sidx, didx, w):
    f32 = jnp.float32
    kern = pl.kernel(
        _sc_body,
        out_type=[
            jax.ShapeDtypeStruct((N_PAD, D), f32),
            jax.ShapeDtypeStruct((N_PAD, D), f32),
        ],
        mesh=plsc.VectorSubcoreMesh(core_axis_name="c", subcore_axis_name="s"),
        scratch_types=[
            pltpu.VMEM((CHUNK,), jnp.int32),
            pltpu.VMEM((CHUNK,), jnp.int32),
            pltpu.VMEM((CHUNK,), f32),
            pltpu.VMEM((CHUNK, D), f32),
            pltpu.VMEM_SHARED((N_PAD, D), f32),
            pltpu.SemaphoreType.DMA,
        ],
    )
    return kern(meff, mx, sidx, didx, w)


def _epi_body(d_ref, h_ref, w_ref, out_ref):
    d = d_ref[...]
    am = jnp.where(d == 0.0, 0.0, 1.0 / d)
    h = h_ref[...] * am
    p = jnp.dot(h, w_ref[...], preferred_element_type=jnp.float32)
    out_ref[...] = jnp.where(p > 0.0, p, jnp.exp(p) - 1.0)


def _epilogue(dsum, hsum, w):
    return pl.pallas_call(
        _epi_body,
        grid=(N_PAD // EPI_BLK,),
        in_specs=[
            pl.BlockSpec((EPI_BLK, D), lambda i: (i, 0)),
            pl.BlockSpec((EPI_BLK, D), lambda i: (i, 0)),
            pl.BlockSpec((D, D), lambda i: (0, 0)),
        ],
        out_specs=pl.BlockSpec((EPI_BLK, D), lambda i: (i, 0)),
        out_shape=jax.ShapeDtypeStruct((N_PAD, D), jnp.float32),
    )(dsum, hsum, w)


@jax.jit
def kernel(x, edge_index, edge_weight, train_fts_id, W, M):
    src = edge_index[0].astype(jnp.int32)
    dst = edge_index[1].astype(jnp.int32)
    w = edge_weight.astype(jnp.float32)

    pad = E_PAD - E
    sidx = jnp.concatenate([src, jnp.zeros((pad,), jnp.int32)])
    didx = jnp.concatenate([dst, jnp.full((pad,), N, jnp.int32)])
    wpad = jnp.concatenate([w, jnp.zeros((pad,), jnp.float32)])

    ids = train_fts_id.astype(jnp.int32)
    ids_pad = jnp.concatenate(
        [ids, jnp.full((NT_PAD - ids.shape[0],), -1, jnp.int32)]
    ).reshape(NT_PAD // 128, 128)

    meff, mx = _prologue(ids_pad, M, x)
    dsum, hsum = _segment_sums(meff, mx, sidx, didx, wpad)
    out = _epilogue(dsum, hsum, W)
    return out[:N]


# speedup vs baseline: 3.9281x; 1.1688x over previous
import functools

import jax
import jax.numpy as jnp
from jax import lax
from jax.experimental import pallas as pl
from jax.experimental.pallas import tpu as pltpu
from jax.experimental.pallas import tpu_sc as plsc

N = 10000
E = 320000
D = 128

NC = 2
NS = 16
CHUNK = 128
NCH = 158
PER_TILE = NCH * CHUNK
E_PAD = PER_TILE * NS
ROWS_PER_TILE = 640
N_PAD = ROWS_PER_TILE * NS

PRO_BLK = 400
EPI_BLK = 512
NT_PAD = 1024


def _pro_body(ids_ref, m_ref, x_ref, meff_ref, mx_ref):
    base = pl.program_id(0) * PRO_BLK
    rows = base + lax.broadcasted_iota(jnp.int32, (PRO_BLK, 1), 0)
    ids = ids_ref[...]
    hit = jnp.zeros((PRO_BLK, 1), dtype=jnp.bool_)
    for j in range(NT_PAD // 128):
        hit = hit | jnp.any(rows == ids[j:j + 1, :], axis=1, keepdims=True)
    meff = jnp.where(hit, 1.0, jax.nn.sigmoid(m_ref[...]))
    meff_ref[...] = meff
    mx_ref[...] = meff * x_ref[...]


def _prologue(train_ids_pad, m, x):
    return pl.pallas_call(
        _pro_body,
        grid=(N // PRO_BLK,),
        in_specs=[
            pl.BlockSpec((NT_PAD // 128, 128), lambda i: (0, 0)),
            pl.BlockSpec((PRO_BLK, D), lambda i: (i, 0)),
            pl.BlockSpec((PRO_BLK, D), lambda i: (i, 0)),
        ],
        out_specs=[
            pl.BlockSpec((PRO_BLK, D), lambda i: (i, 0)),
            pl.BlockSpec((PRO_BLK, D), lambda i: (i, 0)),
        ],
        out_shape=[
            jax.ShapeDtypeStruct((N, D), jnp.float32),
            jax.ShapeDtypeStruct((N, D), jnp.float32),
        ],
    )(train_ids_pad, m, x)


def _sc_body(meff_hbm, mx_hbm, sidx_hbm, didx_hbm, w_hbm, dsum_hbm, hsum_hbm,
             sidx0, sidx1, didx0, didx1, w0, w1, rows0, rows1, acc_sh,
             gsem0, gsem1, ssem0, ssem1):
    cid = lax.axis_index("c")
    tid = lax.axis_index("s")
    tbase = tid * PER_TILE
    sidx = (sidx0, sidx1)
    didx = (didx0, didx1)
    wbuf = (w0, w1)
    rows = (rows0, rows1)
    gsem = (gsem0, gsem1)
    ssem = (ssem0, ssem1)

    def _zero_row(i, _):
        for j in range(D // 16):
            rows0[i, pl.ds(j * 16, 16)] = jnp.zeros((16,), jnp.float32)
        return 0
    lax.fori_loop(0, CHUNK, _zero_row, 0)
    for k in range(ROWS_PER_TILE // CHUNK):
        pltpu.sync_copy(
            rows0, acc_sh.at[pl.ds(tid * ROWS_PER_TILE + k * CHUNK, CHUNK)])
    plsc.subcore_barrier()

    def _run(scaled):
        table = mx_hbm if scaled else meff_hbm

        def _issue(c, b):
            off = tbase + c * CHUNK
            pltpu.sync_copy(sidx_hbm.at[pl.ds(off, CHUNK)], sidx[b])
            pltpu.sync_copy(didx_hbm.at[pl.ds(off, CHUNK)], didx[b])
            if scaled:
                pltpu.sync_copy(w_hbm.at[pl.ds(off, CHUNK)], wbuf[b])
            pltpu.async_copy(table.at[sidx[b]], rows[b], gsem[b])

        def _step(c, b):
            nxt = b ^ 1

            @pl.when(c >= 1)
            def _():
                pltpu.make_async_copy(
                    rows[nxt], acc_sh.at[didx[nxt]], ssem[nxt]).wait()

            @pl.when(c + 1 < NCH)
            def _():
                _issue(c + 1, nxt)

            pltpu.make_async_copy(table.at[sidx[b]], rows[b], gsem[b]).wait()
            if scaled:
                def _scale_grp(g, _):
                    wv = wbuf[b][pl.ds(g * 16, 16)]
                    for l in range(16):
                        wl = wv[l]
                        e = g * 16 + l
                        for j in range(D // 16):
                            sl = pl.ds(j * 16, 16)
                            rows[b][e, sl] = rows[b][e, sl] * wl
                    return 0
                lax.fori_loop(0, CHUNK // 16, _scale_grp, 0)
            pltpu.async_copy(rows[b], acc_sh.at[didx[b]], ssem[b], add=True)

        _issue(0, 0)

        def _pair(g, _):
            _step(g * 2, 0)
            _step(g * 2 + 1, 1)
            return 0
        lax.fori_loop(0, NCH // 2, _pair, 0)
        pltpu.make_async_copy(rows[1], acc_sh.at[didx[1]], ssem[1]).wait()

    @pl.when(cid == 0)
    def _():
        _run(scaled=False)

    @pl.when(cid == 1)
    def _():
        _run(scaled=True)

    plsc.subcore_barrier()

    out_slice = pl.ds(tid * ROWS_PER_TILE, ROWS_PER_TILE)

    @pl.when(cid == 0)
    def _():
        pltpu.sync_copy(acc_sh.at[out_slice], dsum_hbm.at[out_slice])

    @pl.when(cid == 1)
    def _():
        pltpu.sync_copy(acc_sh.at[out_slice], hsum_hbm.at[out_slice])


def _segment_sums(meff, mx, sidx, didx, w):
    f32 = jnp.float32
    kern = pl.kernel(
        _sc_body,
        out_type=[
            jax.ShapeDtypeStruct((N_PAD, D), f32),
            jax.ShapeDtypeStruct((N_PAD, D), f32),
        ],
        mesh=plsc.VectorSubcoreMesh(core_axis_name="c", subcore_axis_name="s"),
        scratch_types=[
            pltpu.VMEM((CHUNK,), jnp.int32),
            pltpu.VMEM((CHUNK,), jnp.int32),
            pltpu.VMEM((CHUNK,), jnp.int32),
            pltpu.VMEM((CHUNK,), jnp.int32),
            pltpu.VMEM((CHUNK,), f32),
            pltpu.VMEM((CHUNK,), f32),
            pltpu.VMEM((CHUNK, D), f32),
            pltpu.VMEM((CHUNK, D), f32),
            pltpu.VMEM_SHARED((N_PAD, D), f32),
            pltpu.SemaphoreType.DMA,
            pltpu.SemaphoreType.DMA,
            pltpu.SemaphoreType.DMA,
            pltpu.SemaphoreType.DMA,
        ],
    )
    return kern(meff, mx, sidx, didx, w)


def _epi_body(d_ref, h_ref, w_ref, out_ref):
    d = d_ref[...]
    am = jnp.where(d == 0.0, 0.0, 1.0 / d)
    h = h_ref[...] * am
    p = jnp.dot(h, w_ref[...], preferred_element_type=jnp.float32)
    out_ref[...] = jnp.where(p > 0.0, p, jnp.exp(p) - 1.0)


def _epilogue(dsum, hsum, w):
    return pl.pallas_call(
        _epi_body,
        grid=(N_PAD // EPI_BLK,),
        in_specs=[
            pl.BlockSpec((EPI_BLK, D), lambda i: (i, 0)),
            pl.BlockSpec((EPI_BLK, D), lambda i: (i, 0)),
            pl.BlockSpec((D, D), lambda i: (0, 0)),
        ],
        out_specs=pl.BlockSpec((EPI_BLK, D), lambda i: (i, 0)),
        out_shape=jax.ShapeDtypeStruct((N_PAD, D), jnp.float32),
    )(dsum, hsum, w)


@jax.jit
def kernel(x, edge_index, edge_weight, train_fts_id, W, M):
    src = edge_index[0].astype(jnp.int32)
    dst = edge_index[1].astype(jnp.int32)
    w = edge_weight.astype(jnp.float32)

    pad = E_PAD - E
    sidx = jnp.concatenate([src, jnp.zeros((pad,), jnp.int32)])
    didx = jnp.concatenate([dst, jnp.full((pad,), N, jnp.int32)])
    wpad = jnp.concatenate([w, jnp.zeros((pad,), jnp.float32)])

    ids = train_fts_id.astype(jnp.int32)
    ids_pad = jnp.concatenate(
        [ids, jnp.full((NT_PAD - ids.shape[0],), -1, jnp.int32)]
    ).reshape(NT_PAD // 128, 128)

    meff, mx = _prologue(ids_pad, M, x)
    dsum, hsum = _segment_sums(meff, mx, sidx, didx, wpad)
    out = _epilogue(dsum, hsum, W)
    return out[:N]
